# baseline (speedup 1.0000x reference)
import jax
import jax.numpy as jnp
from jax import lax
from jax.experimental import pallas as pl
from jax.experimental.pallas import tpu as pltpu
from jax.experimental.pallas import tpu_sc as plsc

N = 10000
NP = 10240
E = 320000
D_IN = 128
HID = 128
NCLS = 64

ROWB = 1024
NCORES = 2
NSUB = 16
CH1 = 80
CH2 = 80
E_PER_TILE_S1 = E // NSUB
E_PER_TILE_S2 = E // (NCORES * NSUB)
ZR = NP // NSUB


def _leaky(v):
    return jnp.where(v > 0, v, 0.2 * v)


def _tc_feats_body(x_ref, w_ref, aa_ref, h_ref, asad_ref, mm_ref):
    i = pl.program_id(0)
    h = jnp.dot(x_ref[...], w_ref[...], preferred_element_type=jnp.float32)
    h_ref[...] = h
    asad = lax.dot_general(aa_ref[...], h, (((1,), (1,)), ((), ())),
                           preferred_element_type=jnp.float32)
    asad_ref[...] = asad

    @pl.when(i == 0)
    def _():
        mm_ref[...] = jnp.full((2, 128), -jnp.inf, jnp.float32)

    cur = jnp.max(asad, axis=1, keepdims=True)
    mm_ref[...] = jnp.maximum(mm_ref[...], jnp.broadcast_to(cur, (2, 128)))


def _tc_feats(x, w, aa, d_out):
    n, d_in = x.shape
    grid = n // ROWB
    return pl.pallas_call(
        _tc_feats_body,
        grid=(grid,),
        in_specs=[
            pl.BlockSpec((ROWB, d_in), lambda i: (i, 0)),
            pl.BlockSpec((d_in, d_out), lambda i: (0, 0)),
            pl.BlockSpec((2, d_out), lambda i: (0, 0)),
        ],
        out_specs=[
            pl.BlockSpec((ROWB, d_out), lambda i: (i, 0)),
            pl.BlockSpec((2, ROWB), lambda i: (0, i)),
            pl.BlockSpec((2, 128), lambda i: (0, 0)),
        ],
        out_shape=[
            jax.ShapeDtypeStruct((n, d_out), jnp.float32),
            jax.ShapeDtypeStruct((2, n), jnp.float32),
            jax.ShapeDtypeStruct((2, 128), jnp.float32),
        ],
    )(x, w, aa)


def _tc_mid_body(pa_ref, pb_ref, da_ref, db_ref, b_ref, w_ref, aa_ref,
                 h_ref, asad_ref, mm_ref):
    i = pl.program_id(0)
    den = da_ref[...] + db_ref[...] + 1e-16
    hin = jnp.maximum((pa_ref[...] + pb_ref[...]) / den + b_ref[...], 0.0)
    h = jnp.dot(hin, w_ref[...], preferred_element_type=jnp.float32)
    h_ref[...] = h
    asad = lax.dot_general(aa_ref[...], h, (((1,), (1,)), ((), ())),
                           preferred_element_type=jnp.float32)
    asad_ref[...] = asad

    @pl.when(i == 0)
    def _():
        mm_ref[...] = jnp.full((2, 128), -jnp.inf, jnp.float32)

    cur = jnp.max(asad, axis=1, keepdims=True)
    mm_ref[...] = jnp.maximum(mm_ref[...], jnp.broadcast_to(cur, (2, 128)))


def _tc_mid(pa, pb, da, db, b, w, aa, d_out):
    n, d_in = pa.shape
    grid = n // ROWB
    return pl.pallas_call(
        _tc_mid_body,
        grid=(grid,),
        in_specs=[
            pl.BlockSpec((ROWB, d_in), lambda i: (i, 0)),
            pl.BlockSpec((ROWB, d_in), lambda i: (i, 0)),
            pl.BlockSpec((ROWB, 1), lambda i: (i, 0)),
            pl.BlockSpec((ROWB, 1), lambda i: (i, 0)),
            pl.BlockSpec((1, d_in), lambda i: (0, 0)),
            pl.BlockSpec((d_in, d_out), lambda i: (0, 0)),
            pl.BlockSpec((2, d_out), lambda i: (0, 0)),
        ],
        out_specs=[
            pl.BlockSpec((ROWB, d_out), lambda i: (i, 0)),
            pl.BlockSpec((2, ROWB), lambda i: (0, i)),
            pl.BlockSpec((2, 128), lambda i: (0, 0)),
        ],
        out_shape=[
            jax.ShapeDtypeStruct((n, d_out), jnp.float32),
            jax.ShapeDtypeStruct((2, n), jnp.float32),
            jax.ShapeDtypeStruct((2, 128), jnp.float32),
        ],
    )(pa, pb, da, db, b, w, aa)


def _tc_fin_body(pa_ref, pb_ref, da_ref, db_ref, b_ref, o_ref):
    d = o_ref.shape[1]
    den = da_ref[...] + db_ref[...] + 1e-16
    o_ref[...] = (pa_ref[:, :d] + pb_ref[:, :d]) / den + b_ref[...]


def _tc_fin(pa, pb, da, db, b):
    n, din = pa.shape
    d = b.shape[1]
    grid = n // ROWB
    return pl.pallas_call(
        _tc_fin_body,
        grid=(grid,),
        in_specs=[
            pl.BlockSpec((ROWB, din), lambda i: (i, 0)),
            pl.BlockSpec((ROWB, din), lambda i: (i, 0)),
            pl.BlockSpec((ROWB, 1), lambda i: (i, 0)),
            pl.BlockSpec((ROWB, 1), lambda i: (i, 0)),
            pl.BlockSpec((1, d), lambda i: (0, 0)),
        ],
        out_specs=pl.BlockSpec((ROWB, d), lambda i: (i, 0)),
        out_shape=jax.ShapeDtypeStruct((n, d), jnp.float32),
    )(pa, pb, da, db, b)


EPT = E // (NCORES * NSUB)
CH = 80
NCH = EPT // CH
BLK = 5
SLABE = BLK * CH
NBLK = NCH // BLK


def _make_sc_layer(d_feat, d_active):
    mesh = plsc.VectorSubcoreMesh(core_axis_name="c", subcore_axis_name="s")

    def body(h_hbm, as_hbm, ad_hbm, mm_hbm, src_hbm, dst_hbm, ew_hbm,
             zrow_hbm, z1_hbm, pa_hbm, pb_hbm, da_hbm, db_hbm,
             as_v, ad_v, mm_v, sidx_v, didx_v, ew_v, ee_v, wv_v,
             rows_v, acc_s, den_s, sem_g, sem_s, sem_i):
        c = lax.axis_index("c")
        s = lax.axis_index("s")
        w = c * NSUB + s
        base_e = w * EPT

        base_c = w * NCH

        def load_slab(o, a):
            off = base_c + o * BLK
            pltpu.async_copy(src_hbm.at[pl.ds(off, BLK)],
                             sidx_v.at[pl.ds(a * BLK, BLK)], sem_i)
            pltpu.async_copy(dst_hbm.at[pl.ds(off, BLK)],
                             didx_v.at[pl.ds(a * BLK, BLK)], sem_i)
            pltpu.async_copy(ew_hbm.at[pl.ds(off, BLK)],
                             ew_v.at[pl.ds(a * BLK, BLK)], sem_i)

        def wait_slab(a):
            pltpu.make_async_copy(src_hbm.at[pl.ds(0, BLK)],
                                  sidx_v.at[pl.ds(a * BLK, BLK)],
                                  sem_i).wait()
            pltpu.make_async_copy(dst_hbm.at[pl.ds(0, BLK)],
                                  didx_v.at[pl.ds(a * BLK, BLK)],
                                  sem_i).wait()
            pltpu.make_async_copy(ew_hbm.at[pl.ds(0, BLK)],
                                  ew_v.at[pl.ds(a * BLK, BLK)],
                                  sem_i).wait()

        def fire_gather(row, b):
            pltpu.async_copy(h_hbm.at[sidx_v.at[row, 0]], rows_v.at[b],
                             sem_g)

        def wait_gather(b):
            pltpu.make_async_copy(h_hbm.at[sidx_v.at[0, 0]], rows_v.at[b],
                                  sem_g).wait()

        pltpu.sync_copy(zrow_hbm, acc_s.at[pl.ds(s * ZR, ZR)])
        pltpu.sync_copy(z1_hbm, den_s.at[pl.ds(s * ZR, ZR)])
        pltpu.sync_copy(as_hbm, as_v)
        pltpu.sync_copy(ad_hbm, ad_v)
        pltpu.sync_copy(mm_hbm, mm_v)
        plsc.subcore_barrier()

        msum = mm_v[0, pl.ds(0, 16)][0] + mm_v[1, pl.ds(0, 16)][0]
        shift = jnp.where(msum > 0, msum, 0.2 * msum)

        load_slab(0, 0)
        wait_slab(0)
        fire_gather(0, 0)
        fire_gather(1, 1)

        def one_chunk(k, buf):
            blk = lax.div(k, BLK)
            pos = lax.rem(k, BLK)
            slab = lax.rem(blk, 2)
            row = slab * BLK + pos

            @pl.when(jnp.logical_and(pos == 0, blk + 1 < NBLK))
            def _():
                load_slab(blk + 1, 1 - slab)

            wait_gather(buf)
            for g in range(CH // 16):
                off = g * 16
                si = sidx_v[row, 0, pl.ds(off, 16)]
                di = didx_v[row, 0, pl.ds(off, 16)]
                e = plsc.load_gather(as_v, [si]) + plsc.load_gather(ad_v, [di])
                e = _leaky(e)
                ex = jnp.exp(e - shift)
                ee_v[0, pl.ds(off, 16)] = ex
                wv = ex * ew_v[row, 0, pl.ds(off, 16)]
                avv = [jnp.broadcast_to(wv[r16], (16,)) for r16 in range(16)]
                for j in range(0, d_active // 16, 2):
                    sl0 = pl.ds(j * 16, 16)
                    sl1 = pl.ds(j * 16 + 16, 16)
                    vals0 = [rows_v[buf, g * 16 + r16, sl0]
                             for r16 in range(16)]
                    vals1 = [rows_v[buf, g * 16 + r16, sl1]
                             for r16 in range(16)]
                    for r16 in range(16):
                        rows_v[buf, g * 16 + r16, sl0] = vals0[r16] * avv[r16]
                    for r16 in range(16):
                        rows_v[buf, g * 16 + r16, sl1] = vals1[r16] * avv[r16]
            d1 = pltpu.async_copy(rows_v.at[buf], acc_s.at[didx_v.at[row, 0]],
                                  sem_s, add=True)
            d2 = pltpu.async_copy(ee_v.at[0], den_s.at[didx_v.at[row, 0]],
                                  sem_s, add=True)
            d1.wait()
            d2.wait()

            @pl.when(jnp.logical_and(pos == 3, blk + 1 < NBLK))
            def _():
                wait_slab(1 - slab)

            nxt = k + 2

            @pl.when(nxt < NCH)
            def _():
                nrow = lax.rem(lax.div(nxt, BLK), 2) * BLK + lax.rem(nxt, BLK)
                fire_gather(nrow, buf)

        def pair_step(m, carry):
            one_chunk(2 * m, 0)
            one_chunk(2 * m + 1, 1)
            return carry

        lax.fori_loop(0, NCH // 2, pair_step, 0)
        one_chunk(NCH - 1, 0)
        plsc.subcore_barrier()

        @pl.when(c == 0)
        def _():
            pltpu.sync_copy(acc_s.at[pl.ds(s * ZR, ZR)],
                            pa_hbm.at[pl.ds(s * ZR, ZR)])
            pltpu.sync_copy(den_s.at[pl.ds(s * ZR, ZR)],
                            da_hbm.at[pl.ds(s * ZR, ZR)])

        @pl.when(c == 1)
        def _():
            pltpu.sync_copy(acc_s.at[pl.ds(s * ZR, ZR)],
                            pb_hbm.at[pl.ds(s * ZR, ZR)])
            pltpu.sync_copy(den_s.at[pl.ds(s * ZR, ZR)],
                            db_hbm.at[pl.ds(s * ZR, ZR)])

    return pl.kernel(
        body,
        out_type=[
            jax.ShapeDtypeStruct((NP, d_feat), jnp.float32),
            jax.ShapeDtypeStruct((NP, d_feat), jnp.float32),
            jax.ShapeDtypeStruct((NP,), jnp.float32),
            jax.ShapeDtypeStruct((NP,), jnp.float32),
        ],
        mesh=mesh,
        compiler_params=pltpu.CompilerParams(needs_layout_passes=False),
        scratch_types=[
            pltpu.VMEM((NP,), jnp.float32),
            pltpu.VMEM((NP,), jnp.float32),
            pltpu.VMEM((2, 128), jnp.float32),
            pltpu.VMEM((2 * BLK, 1, CH), jnp.int32),
            pltpu.VMEM((2 * BLK, 1, CH), jnp.int32),
            pltpu.VMEM((2 * BLK, 1, CH), jnp.float32),
            pltpu.VMEM((1, CH), jnp.float32),
            pltpu.VMEM((16,), jnp.float32),
            pltpu.VMEM((2, CH, d_feat), jnp.float32),
            pltpu.VMEM_SHARED((NP, d_feat), jnp.float32),
            pltpu.VMEM_SHARED((NP,), jnp.float32),
            pltpu.SemaphoreType.DMA,
            pltpu.SemaphoreType.DMA,
            pltpu.SemaphoreType.DMA,
        ],
    )


_sc_layer1 = _make_sc_layer(HID, HID)
_sc_layer2 = _make_sc_layer(128, NCLS)


@jax.jit
def kernel(x, edge_index, edge_weight, W1, a_src1, a_dst1, b1,
           W2, a_src2, a_dst2, b2):
    esrc = edge_index[0].astype(jnp.int32).reshape(E // CH, 1, CH)
    edst = edge_index[1].astype(jnp.int32).reshape(E // CH, 1, CH)
    ew3 = edge_weight.reshape(E // CH, 1, CH)
    aa1 = jnp.concatenate([a_src1, a_dst1], axis=0)
    aa2 = jnp.pad(jnp.concatenate([a_src2, a_dst2], axis=0),
                  ((0, 0), (0, 128 - NCLS)))
    W2p = jnp.pad(W2, ((0, 0), (0, 128 - NCLS)))
    zrow1 = jnp.zeros((ZR, HID), jnp.float32)
    zrow2 = jnp.zeros((ZR, 128), jnp.float32)
    z1 = jnp.zeros((ZR,), jnp.float32)
    xp = jnp.pad(x, ((0, NP - N), (0, 0)))

    h1, asad1, mm1 = _tc_feats(xp, W1, aa1, HID)
    pa1, pb1, da1, db1 = _sc_layer1(h1, asad1[0], asad1[1], mm1, esrc, edst,
                                    ew3, zrow1, z1)
    h2, asad2, mm2 = _tc_mid(pa1, pb1, da1.reshape(NP, 1), db1.reshape(NP, 1),
                             b1.reshape(1, HID), W2p, aa2, 128)
    pa2, pb2, da2, db2 = _sc_layer2(h2, asad2[0], asad2[1], mm2, esrc, edst,
                                    ew3, zrow2, z1)
    out = _tc_fin(pa2, pb2, da2.reshape(NP, 1), db2.reshape(NP, 1),
                  b2.reshape(1, NCLS))
    return out[:N]


# speedup vs baseline: 1.1559x; 1.1559x over previous
import jax
import jax.numpy as jnp
from jax import lax
from jax.experimental import pallas as pl
from jax.experimental.pallas import tpu as pltpu
from jax.experimental.pallas import tpu_sc as plsc

N = 10000
NP = 10240
E = 320000
D_IN = 128
HID = 128
NCLS = 64

ROWB = 1024
NCORES = 2
NSUB = 16
CH1 = 80
CH2 = 80
E_PER_TILE_S1 = E // NSUB
E_PER_TILE_S2 = E // (NCORES * NSUB)
ZR = NP // NSUB


def _leaky(v):
    return jnp.where(v > 0, v, 0.2 * v)


def _tc_feats_body(x_ref, w_ref, aa_ref, h_ref, asad_ref, mm_ref):
    i = pl.program_id(0)
    h = jnp.dot(x_ref[...], w_ref[...], preferred_element_type=jnp.float32)
    h_ref[...] = h
    asad = lax.dot_general(aa_ref[...], h, (((1,), (1,)), ((), ())),
                           preferred_element_type=jnp.float32)
    asad_ref[...] = asad

    @pl.when(i == 0)
    def _():
        mm_ref[...] = jnp.full((2, 128), -jnp.inf, jnp.float32)

    cur = jnp.max(asad, axis=1, keepdims=True)
    mm_ref[...] = jnp.maximum(mm_ref[...], jnp.broadcast_to(cur, (2, 128)))


def _tc_feats(x, w, aa, d_out):
    n, d_in = x.shape
    grid = n // ROWB
    return pl.pallas_call(
        _tc_feats_body,
        grid=(grid,),
        in_specs=[
            pl.BlockSpec((ROWB, d_in), lambda i: (i, 0)),
            pl.BlockSpec((d_in, d_out), lambda i: (0, 0)),
            pl.BlockSpec((2, d_out), lambda i: (0, 0)),
        ],
        out_specs=[
            pl.BlockSpec((ROWB, d_out), lambda i: (i, 0)),
            pl.BlockSpec((2, ROWB), lambda i: (0, i)),
            pl.BlockSpec((2, 128), lambda i: (0, 0)),
        ],
        out_shape=[
            jax.ShapeDtypeStruct((n, d_out), jnp.float32),
            jax.ShapeDtypeStruct((2, n), jnp.float32),
            jax.ShapeDtypeStruct((2, 128), jnp.float32),
        ],
    )(x, w, aa)


def _tc_mid_body(pa_ref, pb_ref, da_ref, db_ref, b_ref, w_ref, aa_ref,
                 h_ref, asad_ref, mm_ref):
    i = pl.program_id(0)
    den = da_ref[...] + db_ref[...] + 1e-16
    hin = jnp.maximum((pa_ref[...] + pb_ref[...]) / den + b_ref[...], 0.0)
    h = jnp.dot(hin, w_ref[...], preferred_element_type=jnp.float32)
    h_ref[...] = h
    asad = lax.dot_general(aa_ref[...], h, (((1,), (1,)), ((), ())),
                           preferred_element_type=jnp.float32)
    asad_ref[...] = asad

    @pl.when(i == 0)
    def _():
        mm_ref[...] = jnp.full((2, 128), -jnp.inf, jnp.float32)

    cur = jnp.max(asad, axis=1, keepdims=True)
    mm_ref[...] = jnp.maximum(mm_ref[...], jnp.broadcast_to(cur, (2, 128)))


def _tc_mid(pa, pb, da, db, b, w, aa, d_out):
    n, d_in = pa.shape
    grid = n // ROWB
    return pl.pallas_call(
        _tc_mid_body,
        grid=(grid,),
        in_specs=[
            pl.BlockSpec((ROWB, d_in), lambda i: (i, 0)),
            pl.BlockSpec((ROWB, d_in), lambda i: (i, 0)),
            pl.BlockSpec((ROWB, 1), lambda i: (i, 0)),
            pl.BlockSpec((ROWB, 1), lambda i: (i, 0)),
            pl.BlockSpec((1, d_in), lambda i: (0, 0)),
            pl.BlockSpec((d_in, d_out), lambda i: (0, 0)),
            pl.BlockSpec((2, d_out), lambda i: (0, 0)),
        ],
        out_specs=[
            pl.BlockSpec((ROWB, d_out), lambda i: (i, 0)),
            pl.BlockSpec((2, ROWB), lambda i: (0, i)),
            pl.BlockSpec((2, 128), lambda i: (0, 0)),
        ],
        out_shape=[
            jax.ShapeDtypeStruct((n, d_out), jnp.float32),
            jax.ShapeDtypeStruct((2, n), jnp.float32),
            jax.ShapeDtypeStruct((2, 128), jnp.float32),
        ],
    )(pa, pb, da, db, b, w, aa)


def _tc_fin_body(pa_ref, pb_ref, da_ref, db_ref, b_ref, o_ref):
    d = o_ref.shape[1]
    den = da_ref[...] + db_ref[...] + 1e-16
    o_ref[...] = (pa_ref[:, :d] + pb_ref[:, :d]) / den + b_ref[...]


def _tc_fin(pa, pb, da, db, b):
    n, din = pa.shape
    d = b.shape[1]
    grid = n // ROWB
    return pl.pallas_call(
        _tc_fin_body,
        grid=(grid,),
        in_specs=[
            pl.BlockSpec((ROWB, din), lambda i: (i, 0)),
            pl.BlockSpec((ROWB, din), lambda i: (i, 0)),
            pl.BlockSpec((ROWB, 1), lambda i: (i, 0)),
            pl.BlockSpec((ROWB, 1), lambda i: (i, 0)),
            pl.BlockSpec((1, d), lambda i: (0, 0)),
        ],
        out_specs=pl.BlockSpec((ROWB, d), lambda i: (i, 0)),
        out_shape=jax.ShapeDtypeStruct((n, d), jnp.float32),
    )(pa, pb, da, db, b)


EPT = E // (NCORES * NSUB)
CH = 80
NCH = EPT // CH
BLK = 5
SLABE = BLK * CH
NBLK = NCH // BLK


def _make_sc_layer(d_feat, d_active):
    mesh = plsc.VectorSubcoreMesh(core_axis_name="c", subcore_axis_name="s")

    def body(h_hbm, as_hbm, ad_hbm, mm_hbm, src_hbm, dst_hbm, ew_hbm,
             zrow_hbm, z1_hbm, pa_hbm, pb_hbm, da_hbm, db_hbm,
             as_v, ad_v, mm_v, sidx_v, didx_v, ew_v, ee_v, wv_v,
             rows_v, acc_s, den_s, sem_g, sem_s, sem_i):
        c = lax.axis_index("c")
        s = lax.axis_index("s")
        w = c * NSUB + s
        base_e = w * EPT

        base_c = w * NCH

        def load_slab(o, a):
            off = base_c + o * BLK
            pltpu.async_copy(src_hbm.at[pl.ds(off, BLK)],
                             sidx_v.at[pl.ds(a * BLK, BLK)], sem_i)
            pltpu.async_copy(dst_hbm.at[pl.ds(off, BLK)],
                             didx_v.at[pl.ds(a * BLK, BLK)], sem_i)
            pltpu.async_copy(ew_hbm.at[pl.ds(off, BLK)],
                             ew_v.at[pl.ds(a * BLK, BLK)], sem_i)

        def wait_slab(a):
            pltpu.make_async_copy(src_hbm.at[pl.ds(0, BLK)],
                                  sidx_v.at[pl.ds(a * BLK, BLK)],
                                  sem_i).wait()
            pltpu.make_async_copy(dst_hbm.at[pl.ds(0, BLK)],
                                  didx_v.at[pl.ds(a * BLK, BLK)],
                                  sem_i).wait()
            pltpu.make_async_copy(ew_hbm.at[pl.ds(0, BLK)],
                                  ew_v.at[pl.ds(a * BLK, BLK)],
                                  sem_i).wait()

        def fire_gather(row, b):
            pltpu.async_copy(h_hbm.at[sidx_v.at[row, 0]], rows_v.at[b],
                             sem_g)

        def wait_gather(b):
            pltpu.make_async_copy(h_hbm.at[sidx_v.at[0, 0]], rows_v.at[b],
                                  sem_g).wait()

        pltpu.sync_copy(zrow_hbm, acc_s.at[pl.ds(s * ZR, ZR)])
        pltpu.sync_copy(z1_hbm, den_s.at[pl.ds(s * ZR, ZR)])
        pltpu.sync_copy(as_hbm, as_v)
        pltpu.sync_copy(ad_hbm, ad_v)
        pltpu.sync_copy(mm_hbm, mm_v)
        plsc.subcore_barrier()

        msum = mm_v[0, pl.ds(0, 16)][0] + mm_v[1, pl.ds(0, 16)][0]
        shift = jnp.where(msum > 0, msum, 0.2 * msum)

        load_slab(0, 0)
        wait_slab(0)
        fire_gather(0, 0)
        fire_gather(1, 1)

        def one_chunk(k, buf):
            blk = lax.div(k, BLK)
            pos = lax.rem(k, BLK)
            slab = lax.rem(blk, 2)
            row = slab * BLK + pos

            @pl.when(jnp.logical_and(pos == 0, blk + 1 < NBLK))
            def _():
                load_slab(blk + 1, 1 - slab)

            wait_gather(buf)
            for g in range(CH // 16):
                off = g * 16
                si = sidx_v[row, 0, pl.ds(off, 16)]
                di = didx_v[row, 0, pl.ds(off, 16)]
                e = plsc.load_gather(as_v, [si]) + plsc.load_gather(ad_v, [di])
                e = _leaky(e)
                ex = jnp.exp(e - shift)
                ee_v[0, pl.ds(off, 16)] = ex
                wv = ex * ew_v[row, 0, pl.ds(off, 16)]

                def row_body(r16, cy):
                    a = lax.gather(
                        wv, jnp.zeros((16, 1), jnp.int32) + r16,
                        lax.GatherDimensionNumbers(
                            offset_dims=(), collapsed_slice_dims=(0,),
                            start_index_map=(0,)),
                        (1,),
                        mode=lax.GatherScatterMode.PROMISE_IN_BOUNDS)
                    rr = g * 16 + r16
                    for j in range(d_active // 16):
                        sl = pl.ds(j * 16, 16)
                        rows_v[buf, rr, sl] = rows_v[buf, rr, sl] * a
                    return cy

                lax.fori_loop(0, 16, row_body, 0)
            d1 = pltpu.async_copy(rows_v.at[buf], acc_s.at[didx_v.at[row, 0]],
                                  sem_s, add=True)
            d2 = pltpu.async_copy(ee_v.at[0], den_s.at[didx_v.at[row, 0]],
                                  sem_s, add=True)
            d1.wait()
            d2.wait()

            @pl.when(jnp.logical_and(pos == 3, blk + 1 < NBLK))
            def _():
                wait_slab(1 - slab)

            nxt = k + 2

            @pl.when(nxt < NCH)
            def _():
                nrow = lax.rem(lax.div(nxt, BLK), 2) * BLK + lax.rem(nxt, BLK)
                fire_gather(nrow, buf)

        def pair_step(m, carry):
            one_chunk(2 * m, 0)
            one_chunk(2 * m + 1, 1)
            return carry

        lax.fori_loop(0, NCH // 2, pair_step, 0)
        one_chunk(NCH - 1, 0)
        plsc.subcore_barrier()

        @pl.when(c == 0)
        def _():
            pltpu.sync_copy(acc_s.at[pl.ds(s * ZR, ZR)],
                            pa_hbm.at[pl.ds(s * ZR, ZR)])
            pltpu.sync_copy(den_s.at[pl.ds(s * ZR, ZR)],
                            da_hbm.at[pl.ds(s * ZR, ZR)])

        @pl.when(c == 1)
        def _():
            pltpu.sync_copy(acc_s.at[pl.ds(s * ZR, ZR)],
                            pb_hbm.at[pl.ds(s * ZR, ZR)])
            pltpu.sync_copy(den_s.at[pl.ds(s * ZR, ZR)],
                            db_hbm.at[pl.ds(s * ZR, ZR)])

    return pl.kernel(
        body,
        out_type=[
            jax.ShapeDtypeStruct((NP, d_feat), jnp.float32),
            jax.ShapeDtypeStruct((NP, d_feat), jnp.float32),
            jax.ShapeDtypeStruct((NP,), jnp.float32),
            jax.ShapeDtypeStruct((NP,), jnp.float32),
        ],
        mesh=mesh,
        compiler_params=pltpu.CompilerParams(needs_layout_passes=False),
        scratch_types=[
            pltpu.VMEM((NP,), jnp.float32),
            pltpu.VMEM((NP,), jnp.float32),
            pltpu.VMEM((2, 128), jnp.float32),
            pltpu.VMEM((2 * BLK, 1, CH), jnp.int32),
            pltpu.VMEM((2 * BLK, 1, CH), jnp.int32),
            pltpu.VMEM((2 * BLK, 1, CH), jnp.float32),
            pltpu.VMEM((1, CH), jnp.float32),
            pltpu.VMEM((16,), jnp.float32),
            pltpu.VMEM((2, CH, d_feat), jnp.float32),
            pltpu.VMEM_SHARED((NP, d_feat), jnp.float32),
            pltpu.VMEM_SHARED((NP,), jnp.float32),
            pltpu.SemaphoreType.DMA,
            pltpu.SemaphoreType.DMA,
            pltpu.SemaphoreType.DMA,
        ],
    )


_sc_layer1 = _make_sc_layer(HID, HID)
_sc_layer2 = _make_sc_layer(128, NCLS)


@jax.jit
def kernel(x, edge_index, edge_weight, W1, a_src1, a_dst1, b1,
           W2, a_src2, a_dst2, b2):
    esrc = edge_index[0].astype(jnp.int32).reshape(E // CH, 1, CH)
    edst = edge_index[1].astype(jnp.int32).reshape(E // CH, 1, CH)
    ew3 = edge_weight.reshape(E // CH, 1, CH)
    aa1 = jnp.concatenate([a_src1, a_dst1], axis=0)
    aa2 = jnp.pad(jnp.concatenate([a_src2, a_dst2], axis=0),
                  ((0, 0), (0, 128 - NCLS)))
    W2p = jnp.pad(W2, ((0, 0), (0, 128 - NCLS)))
    zrow1 = jnp.zeros((ZR, HID), jnp.float32)
    zrow2 = jnp.zeros((ZR, 128), jnp.float32)
    z1 = jnp.zeros((ZR,), jnp.float32)
    xp = jnp.pad(x, ((0, NP - N), (0, 0)))

    h1, asad1, mm1 = _tc_feats(xp, W1, aa1, HID)
    pa1, pb1, da1, db1 = _sc_layer1(h1, asad1[0], asad1[1], mm1, esrc, edst,
                                    ew3, zrow1, z1)
    h2, asad2, mm2 = _tc_mid(pa1, pb1, da1.reshape(NP, 1), db1.reshape(NP, 1),
                             b1.reshape(1, HID), W2p, aa2, 128)
    pa2, pb2, da2, db2 = _sc_layer2(h2, asad2[0], asad2[1], mm2, esrc, edst,
                                    ew3, zrow2, z1)
    out = _tc_fin(pa2, pb2, da2.reshape(NP, 1), db2.reshape(NP, 1),
                  b2.reshape(1, NCLS))
    return out[:N]


# speedup vs baseline: 1.1845x; 1.0247x over previous
import jax
import jax.numpy as jnp
from jax import lax
from jax.experimental import pallas as pl
from jax.experimental.pallas import tpu as pltpu
from jax.experimental.pallas import tpu_sc as plsc

N = 10000
NP = 10240
E = 320000
D_IN = 128
HID = 128
NCLS = 64

ROWB = 1024
NCORES = 2
NSUB = 16
CH1 = 80
CH2 = 80
E_PER_TILE_S1 = E // NSUB
E_PER_TILE_S2 = E // (NCORES * NSUB)
ZR = NP // NSUB


def _leaky(v):
    return jnp.where(v > 0, v, 0.2 * v)


def _tc_feats_body(x_ref, w_ref, aa_ref, h_ref, asad_ref, mm_ref):
    i = pl.program_id(0)
    h = jnp.dot(x_ref[...], w_ref[...], preferred_element_type=jnp.float32)
    h_ref[...] = h
    asad = lax.dot_general(aa_ref[...], h, (((1,), (1,)), ((), ())),
                           preferred_element_type=jnp.float32)
    asad_ref[...] = asad

    @pl.when(i == 0)
    def _():
        mm_ref[...] = jnp.full((2, 128), -jnp.inf, jnp.float32)

    cur = jnp.max(asad, axis=1, keepdims=True)
    mm_ref[...] = jnp.maximum(mm_ref[...], jnp.broadcast_to(cur, (2, 128)))


def _tc_feats(x, w, aa, d_out):
    n, d_in = x.shape
    grid = n // ROWB
    return pl.pallas_call(
        _tc_feats_body,
        grid=(grid,),
        in_specs=[
            pl.BlockSpec((ROWB, d_in), lambda i: (i, 0)),
            pl.BlockSpec((d_in, d_out), lambda i: (0, 0)),
            pl.BlockSpec((2, d_out), lambda i: (0, 0)),
        ],
        out_specs=[
            pl.BlockSpec((ROWB, d_out), lambda i: (i, 0)),
            pl.BlockSpec((2, ROWB), lambda i: (0, i)),
            pl.BlockSpec((2, 128), lambda i: (0, 0)),
        ],
        out_shape=[
            jax.ShapeDtypeStruct((n, d_out), jnp.float32),
            jax.ShapeDtypeStruct((2, n), jnp.float32),
            jax.ShapeDtypeStruct((2, 128), jnp.float32),
        ],
    )(x, w, aa)


def _tc_mid_body(pa_ref, pb_ref, da_ref, db_ref, b_ref, w_ref, aa_ref,
                 h_ref, asad_ref, mm_ref):
    i = pl.program_id(0)
    den = da_ref[...] + db_ref[...] + 1e-16
    hin = jnp.maximum((pa_ref[...] + pb_ref[...]) / den + b_ref[...], 0.0)
    h = jnp.dot(hin, w_ref[...], preferred_element_type=jnp.float32)
    h_ref[...] = h
    asad = lax.dot_general(aa_ref[...], h, (((1,), (1,)), ((), ())),
                           preferred_element_type=jnp.float32)
    asad_ref[...] = asad

    @pl.when(i == 0)
    def _():
        mm_ref[...] = jnp.full((2, 128), -jnp.inf, jnp.float32)

    cur = jnp.max(asad, axis=1, keepdims=True)
    mm_ref[...] = jnp.maximum(mm_ref[...], jnp.broadcast_to(cur, (2, 128)))


def _tc_mid(pa, pb, da, db, b, w, aa, d_out):
    n, d_in = pa.shape
    grid = n // ROWB
    return pl.pallas_call(
        _tc_mid_body,
        grid=(grid,),
        in_specs=[
            pl.BlockSpec((ROWB, d_in), lambda i: (i, 0)),
            pl.BlockSpec((ROWB, d_in), lambda i: (i, 0)),
            pl.BlockSpec((ROWB, 1), lambda i: (i, 0)),
            pl.BlockSpec((ROWB, 1), lambda i: (i, 0)),
            pl.BlockSpec((1, d_in), lambda i: (0, 0)),
            pl.BlockSpec((d_in, d_out), lambda i: (0, 0)),
            pl.BlockSpec((2, d_out), lambda i: (0, 0)),
        ],
        out_specs=[
            pl.BlockSpec((ROWB, d_out), lambda i: (i, 0)),
            pl.BlockSpec((2, ROWB), lambda i: (0, i)),
            pl.BlockSpec((2, 128), lambda i: (0, 0)),
        ],
        out_shape=[
            jax.ShapeDtypeStruct((n, d_out), jnp.float32),
            jax.ShapeDtypeStruct((2, n), jnp.float32),
            jax.ShapeDtypeStruct((2, 128), jnp.float32),
        ],
    )(pa, pb, da, db, b, w, aa)


def _tc_fin_body(pa_ref, pb_ref, da_ref, db_ref, b_ref, o_ref):
    d = o_ref.shape[1]
    den = da_ref[...] + db_ref[...] + 1e-16
    o_ref[...] = (pa_ref[:, :d] + pb_ref[:, :d]) / den + b_ref[...]


def _tc_fin(pa, pb, da, db, b):
    n, din = pa.shape
    d = b.shape[1]
    grid = n // ROWB
    return pl.pallas_call(
        _tc_fin_body,
        grid=(grid,),
        in_specs=[
            pl.BlockSpec((ROWB, din), lambda i: (i, 0)),
            pl.BlockSpec((ROWB, din), lambda i: (i, 0)),
            pl.BlockSpec((ROWB, 1), lambda i: (i, 0)),
            pl.BlockSpec((ROWB, 1), lambda i: (i, 0)),
            pl.BlockSpec((1, d), lambda i: (0, 0)),
        ],
        out_specs=pl.BlockSpec((ROWB, d), lambda i: (i, 0)),
        out_shape=jax.ShapeDtypeStruct((n, d), jnp.float32),
    )(pa, pb, da, db, b)


EPT = E // (NCORES * NSUB)
CH = 80
NCH = EPT // CH
BLK = 5
SLABE = BLK * CH
NBLK = NCH // BLK


def _make_sc_layer(d_feat, d_active):
    mesh = plsc.VectorSubcoreMesh(core_axis_name="c", subcore_axis_name="s")

    def body(h_hbm, as_hbm, ad_hbm, mm_hbm, src_hbm, dst_hbm, ew_hbm,
             zrow_hbm, z1_hbm, pa_hbm, pb_hbm, da_hbm, db_hbm,
             as_v, ad_v, mm_v, sidx_v, didx_v, ew_v, ee_v, wv_v,
             rows_v, acc_s, den_s, sem_g, sem_s, sem_i):
        c = lax.axis_index("c")
        s = lax.axis_index("s")
        w = c * NSUB + s
        base_e = w * EPT

        base_c = w * NCH

        def load_slab(o, a):
            off = base_c + o * BLK
            pltpu.async_copy(src_hbm.at[pl.ds(off, BLK)],
                             sidx_v.at[pl.ds(a * BLK, BLK)], sem_i)
            pltpu.async_copy(dst_hbm.at[pl.ds(off, BLK)],
                             didx_v.at[pl.ds(a * BLK, BLK)], sem_i)
            pltpu.async_copy(ew_hbm.at[pl.ds(off, BLK)],
                             ew_v.at[pl.ds(a * BLK, BLK)], sem_i)

        def wait_slab(a):
            pltpu.make_async_copy(src_hbm.at[pl.ds(0, BLK)],
                                  sidx_v.at[pl.ds(a * BLK, BLK)],
                                  sem_i).wait()
            pltpu.make_async_copy(dst_hbm.at[pl.ds(0, BLK)],
                                  didx_v.at[pl.ds(a * BLK, BLK)],
                                  sem_i).wait()
            pltpu.make_async_copy(ew_hbm.at[pl.ds(0, BLK)],
                                  ew_v.at[pl.ds(a * BLK, BLK)],
                                  sem_i).wait()

        def fire_gather(row, b):
            pltpu.async_copy(h_hbm.at[sidx_v.at[row, 0]], rows_v.at[b],
                             sem_g)

        def wait_gather(b):
            pltpu.make_async_copy(h_hbm.at[sidx_v.at[0, 0]], rows_v.at[b],
                                  sem_g).wait()

        pltpu.sync_copy(zrow_hbm, acc_s.at[pl.ds(s * ZR, ZR)])
        pltpu.sync_copy(z1_hbm, den_s.at[pl.ds(s * ZR, ZR)])
        pltpu.sync_copy(as_hbm, as_v)
        pltpu.sync_copy(ad_hbm, ad_v)
        pltpu.sync_copy(mm_hbm, mm_v)
        plsc.subcore_barrier()

        msum = mm_v[0, pl.ds(0, 16)][0] + mm_v[1, pl.ds(0, 16)][0]
        shift = jnp.where(msum > 0, msum, 0.2 * msum)

        load_slab(0, 0)
        wait_slab(0)
        fire_gather(0, 0)
        fire_gather(1, 1)

        def one_chunk(k, buf):
            blk = lax.div(k, BLK)
            pos = lax.rem(k, BLK)
            slab = lax.rem(blk, 2)
            row = slab * BLK + pos

            @pl.when(jnp.logical_and(pos == 0, blk + 1 < NBLK))
            def _():
                load_slab(blk + 1, 1 - slab)

            wait_gather(buf)
            for g in range(CH // 16):
                off = g * 16
                si = sidx_v[row, 0, pl.ds(off, 16)]
                di = didx_v[row, 0, pl.ds(off, 16)]
                e = plsc.load_gather(as_v, [si]) + plsc.load_gather(ad_v, [di])
                e = _leaky(e)
                ex = jnp.exp(e - shift)
                ee_v[0, pl.ds(off, 16)] = ex
                wv = ex * ew_v[row, 0, pl.ds(off, 16)]

                def row_body(rh, cy):
                    dn = lax.GatherDimensionNumbers(
                        offset_dims=(), collapsed_slice_dims=(0,),
                        start_index_map=(0,))
                    pb = lax.GatherScatterMode.PROMISE_IN_BOUNDS
                    r0 = 2 * rh
                    a0 = lax.gather(wv, jnp.zeros((16, 1), jnp.int32) + r0,
                                    dn, (1,), mode=pb)
                    r1 = r0 + 1
                    a1 = lax.gather(wv, jnp.zeros((16, 1), jnp.int32) + r1,
                                    dn, (1,), mode=pb)
                    rr = g * 16 + r0
                    for j in range(d_active // 16):
                        sl = pl.ds(j * 16, 16)
                        rows_v[buf, rr, sl] = rows_v[buf, rr, sl] * a0
                        rows_v[buf, rr + 1, sl] = rows_v[buf, rr + 1, sl] * a1
                    return cy

                lax.fori_loop(0, 8, row_body, 0)
            d1 = pltpu.async_copy(rows_v.at[buf], acc_s.at[didx_v.at[row, 0]],
                                  sem_s, add=True)
            d2 = pltpu.async_copy(ee_v.at[0], den_s.at[didx_v.at[row, 0]],
                                  sem_s, add=True)
            d1.wait()
            d2.wait()

            @pl.when(jnp.logical_and(pos == 3, blk + 1 < NBLK))
            def _():
                wait_slab(1 - slab)

            nxt = k + 2

            @pl.when(nxt < NCH)
            def _():
                nrow = lax.rem(lax.div(nxt, BLK), 2) * BLK + lax.rem(nxt, BLK)
                fire_gather(nrow, buf)

        def pair_step(m, carry):
            one_chunk(2 * m, 0)
            one_chunk(2 * m + 1, 1)
            return carry

        lax.fori_loop(0, NCH // 2, pair_step, 0)
        one_chunk(NCH - 1, 0)
        plsc.subcore_barrier()

        @pl.when(c == 0)
        def _():
            pltpu.sync_copy(acc_s.at[pl.ds(s * ZR, ZR)],
                            pa_hbm.at[pl.ds(s * ZR, ZR)])
            pltpu.sync_copy(den_s.at[pl.ds(s * ZR, ZR)],
                            da_hbm.at[pl.ds(s * ZR, ZR)])

        @pl.when(c == 1)
        def _():
            pltpu.sync_copy(acc_s.at[pl.ds(s * ZR, ZR)],
                            pb_hbm.at[pl.ds(s * ZR, ZR)])
            pltpu.sync_copy(den_s.at[pl.ds(s * ZR, ZR)],
                            db_hbm.at[pl.ds(s * ZR, ZR)])

    return pl.kernel(
        body,
        out_type=[
            jax.ShapeDtypeStruct((NP, d_feat), jnp.float32),
            jax.ShapeDtypeStruct((NP, d_feat), jnp.float32),
            jax.ShapeDtypeStruct((NP,), jnp.float32),
            jax.ShapeDtypeStruct((NP,), jnp.float32),
        ],
        mesh=mesh,
        compiler_params=pltpu.CompilerParams(needs_layout_passes=False),
        scratch_types=[
            pltpu.VMEM((NP,), jnp.float32),
            pltpu.VMEM((NP,), jnp.float32),
            pltpu.VMEM((2, 128), jnp.float32),
            pltpu.VMEM((2 * BLK, 1, CH), jnp.int32),
            pltpu.VMEM((2 * BLK, 1, CH), jnp.int32),
            pltpu.VMEM((2 * BLK, 1, CH), jnp.float32),
            pltpu.VMEM((1, CH), jnp.float32),
            pltpu.VMEM((16,), jnp.float32),
            pltpu.VMEM((2, CH, d_feat), jnp.float32),
            pltpu.VMEM_SHARED((NP, d_feat), jnp.float32),
            pltpu.VMEM_SHARED((NP,), jnp.float32),
            pltpu.SemaphoreType.DMA,
            pltpu.SemaphoreType.DMA,
            pltpu.SemaphoreType.DMA,
        ],
    )


_sc_layer1 = _make_sc_layer(HID, HID)
_sc_layer2 = _make_sc_layer(128, NCLS)


@jax.jit
def kernel(x, edge_index, edge_weight, W1, a_src1, a_dst1, b1,
           W2, a_src2, a_dst2, b2):
    esrc = edge_index[0].astype(jnp.int32).reshape(E // CH, 1, CH)
    edst = edge_index[1].astype(jnp.int32).reshape(E // CH, 1, CH)
    ew3 = edge_weight.reshape(E // CH, 1, CH)
    aa1 = jnp.concatenate([a_src1, a_dst1], axis=0)
    aa2 = jnp.pad(jnp.concatenate([a_src2, a_dst2], axis=0),
                  ((0, 0), (0, 128 - NCLS)))
    W2p = jnp.pad(W2, ((0, 0), (0, 128 - NCLS)))
    zrow1 = jnp.zeros((ZR, HID), jnp.float32)
    zrow2 = jnp.zeros((ZR, 128), jnp.float32)
    z1 = jnp.zeros((ZR,), jnp.float32)
    xp = jnp.pad(x, ((0, NP - N), (0, 0)))

    h1, asad1, mm1 = _tc_feats(xp, W1, aa1, HID)
    pa1, pb1, da1, db1 = _sc_layer1(h1, asad1[0], asad1[1], mm1, esrc, edst,
                                    ew3, zrow1, z1)
    h2, asad2, mm2 = _tc_mid(pa1, pb1, da1.reshape(NP, 1), db1.reshape(NP, 1),
                             b1.reshape(1, HID), W2p, aa2, 128)
    pa2, pb2, da2, db2 = _sc_layer2(h2, asad2[0], asad2[1], mm2, esrc, edst,
                                    ew3, zrow2, z1)
    out = _tc_fin(pa2, pb2, da2.reshape(NP, 1), db2.reshape(NP, 1),
                  b2.reshape(1, NCLS))
    return out[:N]
